# baseline (speedup 1.0000x reference)
import functools

import jax
import jax.numpy as jnp
import numpy as np
from jax.experimental import pallas as pl
from jax.experimental.pallas import tpu as pltpu

BS = 8
NQ = 900
NC = 92
TPI = 25
TT = BS * TPI
PAD = 32
PCOLS = BS * PAD

_SQRT2 = np.float32(np.sqrt(np.float32(2.0)))
_DEN = np.float32(2.5) * _SQRT2
_HN = np.float32(0.3) * _SQRT2
_NEG = np.float32(-3.0e38)
_MARK = np.float32(-1e-7)


def _kernel(logits_ref, pb_ref, tb_ref, lab_ref, out_ref, B_ref):
    for b in range(BS):
        x = logits_ref[b]
        m = jnp.max(x, axis=1, keepdims=True)
        e = jnp.exp(x - m)
        s = jnp.sum(e, axis=1, keepdims=True)
        prob = e / s

        lab = lab_ref[b]
        onehot = (lab == jax.lax.broadcasted_iota(jnp.int32, (PAD, NC), 1)
                  ).astype(jnp.float32)
        clsprob = jax.lax.dot_general(
            onehot, prob, (((1,), (1,)), ((), ())),
            preferred_element_type=jnp.float32)

        pb = pb_ref[b]
        qcx, qcy, qw, qh = pb[0:1], pb[1:2], pb[2:3], pb[3:4]
        tb = tb_ref[b]
        tcx, tcy, tw, th = tb[:, 0:1], tb[:, 1:2], tb[:, 2:3], tb[:, 3:4]

        cbbox = (jnp.abs(qcx - tcx) + jnp.abs(qcy - tcy)
                 + jnp.abs(qw - tw) + jnp.abs(qh - th))

        qx0, qx1 = qcx - 0.5 * qw, qcx + 0.5 * qw
        qy0, qy1 = qcy - 0.5 * qh, qcy + 0.5 * qh
        tx0, tx1 = tcx - 0.5 * tw, tcx + 0.5 * tw
        ty0, ty1 = tcy - 0.5 * th, tcy + 0.5 * th

        area_q = (qx1 - qx0) * (qy1 - qy0)
        area_t = (tx1 - tx0) * (ty1 - ty0)
        iw = jnp.clip(jnp.minimum(qx1, tx1) - jnp.maximum(qx0, tx0), 0.0, None)
        ih = jnp.clip(jnp.minimum(qy1, ty1) - jnp.maximum(qy0, ty0), 0.0, None)
        inter = iw * ih
        union = area_q + area_t - inter
        iou = inter / union
        ew = jnp.clip(jnp.maximum(qx1, tx1) - jnp.minimum(qx0, tx0), 0.0, None)
        eh = jnp.clip(jnp.maximum(qy1, ty1) - jnp.minimum(qy0, ty0), 0.0, None)
        earea = ew * eh
        giou = iou - (earea - union) / earea

        C = 5.0 * cbbox + 1.0 * (-clsprob) + 2.0 * (-giou)

        z = (-C - (-5.5)) / _DEN
        p = 0.5 * (1.0 + jax.lax.erf(z))
        fxa = _HN * jax.lax.erf_inv(p)
        B_ref[PAD * b:PAD * (b + 1), :] = fxa

    col_iota = jax.lax.broadcasted_iota(jnp.int32, (PCOLS, 1), 0)
    q_iota = jax.lax.broadcasted_iota(jnp.int32, (PCOLS, NQ), 1)
    pad_col = (col_iota % PAD) >= TPI

    B = B_ref[...]
    Bc = jnp.minimum(B, 100.0)
    max1 = jnp.max(Bc, axis=1, keepdims=True)
    amax_c = jnp.min(jnp.where(Bc == max1, q_iota, NQ),
                     axis=1, keepdims=True)
    second = jnp.max(jnp.where(q_iota == amax_c, _NEG, Bc),
                     axis=1, keepdims=True)
    gap_col = jnp.where(pad_col, -1.0, max1 - second)
    maxu = jnp.max(B, axis=1, keepdims=True)
    amax_u = jnp.min(jnp.where(B == maxu, q_iota, NQ),
                     axis=1, keepdims=True)

    eye = (jax.lax.broadcasted_iota(jnp.int32, (PCOLS, PCOLS), 0)
           == jax.lax.broadcasted_iota(jnp.int32, (PCOLS, PCOLS), 1)
           ).astype(jnp.float32)
    gap0 = jax.lax.dot_general(gap_col, eye, (((0,), (0,)), ((), ())),
                               preferred_element_type=jnp.float32)

    lane256 = jax.lax.broadcasted_iota(jnp.int32, (1, PCOLS), 1)
    q_lane = jax.lax.broadcasted_iota(jnp.int32, (1, NQ), 1)

    def gap_of(rowvec):
        c = jnp.minimum(rowvec, 100.0)
        m1 = jnp.max(c)
        a1 = jnp.min(jnp.where(c == m1, q_lane, NQ))
        s = jnp.max(jnp.where(q_lane == a1, _NEG, c))
        return m1 - s

    def body(_, carry):
        gap, marker = carry
        gmax = jnp.max(gap)
        col = jnp.min(jnp.where(gap == gmax, lane256, PCOLS))

        v0r = B_ref[0:1, :]
        v1r = B_ref[1:2, :]
        m0 = jnp.max(v0r)
        a0 = jnp.min(jnp.where(v0r == m0, q_lane, NQ))
        m1u = jnp.max(v1r)
        a1u = jnp.min(jnp.where(v1r == m1u, q_lane, NQ))
        mcol = jnp.min(jnp.where(lane256 == col, marker, NQ))
        rstat = jnp.min(jnp.where(col_iota == col, amax_u, NQ))
        resel = jnp.where(mcol == 0, 1, 0)
        rother = jnp.where(mcol >= 0, resel, rstat)
        row = jnp.where(col == 0, a0,
                        jnp.where(col == 1, a1u, rother))

        rm = q_lane == row
        nv0 = jnp.where(col == 0, jnp.where(rm, _MARK, 0.0),
                        jnp.where(rm, 0.0, v0r))
        nv1 = jnp.where(col == 1, jnp.where(rm, _MARK, 0.0),
                        jnp.where(rm, 0.0, v1r))
        B_ref[0:1, :] = nv0
        B_ref[1:2, :] = nv1

        marker = jnp.where(lane256 == col, -1, marker)
        marker = jnp.where((lane256 <= 1) & (marker == row), -1, marker)
        marker = jnp.where(lane256 == col, row, marker)

        g0 = gap_of(nv0)
        g1 = gap_of(nv1)
        gap = jnp.where(lane256 == 0, g0, gap)
        gap = jnp.where(lane256 == 1, g1, gap)
        gap = jnp.where(lane256 == col, 0.0, gap)
        return gap, marker

    marker0 = jnp.full((1, PCOLS), -1, dtype=jnp.int32)
    _, marker = jax.lax.fori_loop(0, TT, body, (gap0, marker0))

    q_sub = jax.lax.broadcasted_iota(jnp.int32, (NQ, PCOLS), 0)
    out_ref[...] = (q_sub == marker).astype(jnp.uint8)


@jax.jit
def kernel(pred_logits, pred_boxes, tgt_labels, tgt_boxes):
    pb = jnp.transpose(pred_boxes, (0, 2, 1))
    tb = tgt_boxes.reshape(BS, TPI, 4)
    tb = jnp.pad(tb, ((0, 0), (0, PAD - TPI), (0, 0)))
    lab = tgt_labels.reshape(BS, TPI)
    lab = jnp.pad(lab, ((0, 0), (0, PAD - TPI)))[..., None]

    out = pl.pallas_call(
        _kernel,
        out_shape=jax.ShapeDtypeStruct((NQ, PCOLS), jnp.uint8),
        scratch_shapes=[pltpu.VMEM((PCOLS, NQ), jnp.float32)],
    )(pred_logits, pb, tb, lab)

    sel = (np.arange(TT) // TPI) * PAD + (np.arange(TT) % TPI)
    return out[:, sel].astype(jnp.bool_)


# speedup vs baseline: 1.5224x; 1.5224x over previous
import functools

import jax
import jax.numpy as jnp
import numpy as np
from jax import lax
from jax.experimental import pallas as pl
from jax.experimental.pallas import tpu as pltpu
from jax.experimental.pallas import tpu_sc as plsc

BS = 8
NQ = 900
NC = 92
TPI = 25
TT = BS * TPI
PAD = 32
PCOLS = BS * PAD
NQP = 912

_SQRT2 = np.float32(np.sqrt(np.float32(2.0)))
_DEN = np.float32(2.5) * _SQRT2
_HN = np.float32(0.3) * _SQRT2
_NEG = np.float32(-3.0e38)
_MARK = np.float32(-1e-7)
_BIGI = np.int32(2147483647)


def _phase_a(logits_ref, pb_ref, tb_ref, lab_ref,
             gap_ref, row_ref, v0_ref, v1_ref, B_ref):
    for b in range(BS):
        x = logits_ref[b]
        m = jnp.max(x, axis=1, keepdims=True)
        e = jnp.exp(x - m)
        s = jnp.sum(e, axis=1, keepdims=True)
        prob = e / s

        lab = lab_ref[b]
        onehot = (lab == jax.lax.broadcasted_iota(jnp.int32, (PAD, NC), 1)
                  ).astype(jnp.float32)
        clsprob = jax.lax.dot_general(
            onehot, prob, (((1,), (1,)), ((), ())),
            preferred_element_type=jnp.float32)

        pb = pb_ref[b]
        qcx, qcy, qw, qh = pb[0:1], pb[1:2], pb[2:3], pb[3:4]
        tb = tb_ref[b]
        tcx, tcy, tw, th = tb[:, 0:1], tb[:, 1:2], tb[:, 2:3], tb[:, 3:4]

        cbbox = (jnp.abs(qcx - tcx) + jnp.abs(qcy - tcy)
                 + jnp.abs(qw - tw) + jnp.abs(qh - th))

        qx0, qx1 = qcx - 0.5 * qw, qcx + 0.5 * qw
        qy0, qy1 = qcy - 0.5 * qh, qcy + 0.5 * qh
        tx0, tx1 = tcx - 0.5 * tw, tcx + 0.5 * tw
        ty0, ty1 = tcy - 0.5 * th, tcy + 0.5 * th

        area_q = (qx1 - qx0) * (qy1 - qy0)
        area_t = (tx1 - tx0) * (ty1 - ty0)
        iw = jnp.clip(jnp.minimum(qx1, tx1) - jnp.maximum(qx0, tx0), 0.0, None)
        ih = jnp.clip(jnp.minimum(qy1, ty1) - jnp.maximum(qy0, ty0), 0.0, None)
        inter = iw * ih
        union = area_q + area_t - inter
        iou = inter / union
        ew = jnp.clip(jnp.maximum(qx1, tx1) - jnp.minimum(qx0, tx0), 0.0, None)
        eh = jnp.clip(jnp.maximum(qy1, ty1) - jnp.minimum(qy0, ty0), 0.0, None)
        earea = ew * eh
        giou = iou - (earea - union) / earea

        C = 5.0 * cbbox + 1.0 * (-clsprob) + 2.0 * (-giou)

        z = (-C - (-5.5)) / _DEN
        p = 0.5 * (1.0 + jax.lax.erf(z))
        fxa = _HN * jax.lax.erf_inv(p)
        B_ref[PAD * b:PAD * (b + 1), :] = fxa

    col_iota = jax.lax.broadcasted_iota(jnp.int32, (PCOLS, 1), 0)
    q_iota = jax.lax.broadcasted_iota(jnp.int32, (PCOLS, NQ), 1)
    pad_col = (col_iota % PAD) >= TPI

    B = B_ref[...]
    Bc = jnp.minimum(B, 100.0)
    max1 = jnp.max(Bc, axis=1, keepdims=True)
    amax_c = jnp.min(jnp.where(Bc == max1, q_iota, NQ),
                     axis=1, keepdims=True)
    second = jnp.max(jnp.where(q_iota == amax_c, _NEG, Bc),
                     axis=1, keepdims=True)
    gap_ref[...] = jnp.where(pad_col, -1.0, max1 - second)
    maxu = jnp.max(B, axis=1, keepdims=True)
    row_ref[...] = jnp.min(jnp.where(B == maxu, q_iota, NQ),
                           axis=1, keepdims=True)
    v0_ref[...] = B_ref[0:1, :]
    v1_ref[...] = B_ref[1:2, :]


def _emit(mark_ref, out_ref):
    q_sub = jax.lax.broadcasted_iota(jnp.int32, (NQ, PCOLS), 0)
    out_ref[...] = (q_sub == mark_ref[...]).astype(jnp.uint8)


def _i16():
    return jax.lax.broadcasted_iota(jnp.int32, (16,), 0)


def _extract_f(chunk, lane):
    return jnp.max(jnp.where(_i16() == lane, chunk, _NEG))


def _extract_i(chunk, lane):
    return jnp.max(jnp.where(_i16() == lane, chunk,
                             jnp.int32(-2147483647)))


def _sc_loop(gap_hbm, row_hbm, v0_hbm, v1_hbm, mark_hbm,
             gap_v, row_v, v0_v, v1_v, mark_v):
    @pl.when((lax.axis_index("c") == 0) & (lax.axis_index("s") == 0))
    def _():
        pltpu.sync_copy(gap_hbm, gap_v)
        pltpu.sync_copy(row_hbm, row_v)
        pltpu.sync_copy(v0_hbm, v0_v)
        pltpu.sync_copy(v1_hbm, v1_v)

        neg1 = jnp.full((16,), -1, jnp.int32)

        def initm(i, carry):
            mark_v[pl.ds(i * 16, 16)] = neg1
            return carry
        lax.fori_loop(0, PCOLS // 16, initm, 0)

        def sweep_stats(vref):
            def p1(i, carry):
                mu, iu, mc, ic = carry
                c = vref[pl.ds(i * 16, 16)]
                cc = jnp.minimum(c, 100.0)
                idx = _i16() + i * 16
                bu = c > mu
                mu = jnp.where(bu, c, mu)
                iu = jnp.where(bu, idx, iu)
                bc = cc > mc
                mc = jnp.where(bc, cc, mc)
                ic = jnp.where(bc, idx, ic)
                return mu, iu, mc, ic
            init = (jnp.full((16,), _NEG), jnp.full((16,), _BIGI),
                    jnp.full((16,), _NEG), jnp.full((16,), _BIGI))
            mu, iu, mc, ic = lax.fori_loop(0, NQP // 16, p1, init)
            m_u = jnp.max(mu)
            a_u = jnp.min(jnp.where(mu == m_u, iu, _BIGI))
            m_c = jnp.max(mc)
            a_c = jnp.min(jnp.where(mc == m_c, ic, _BIGI))

            def p2(i, sv):
                c = jnp.minimum(vref[pl.ds(i * 16, 16)], 100.0)
                idx = _i16() + i * 16
                c = jnp.where(idx == a_c, _NEG, c)
                return jnp.maximum(sv, c)
            sv = lax.fori_loop(0, NQP // 16, p2, jnp.full((16,), _NEG))
            s_c = jnp.max(sv)
            return m_u, a_u, m_c, a_c, s_c

        st0 = sweep_stats(v0_v)
        st1 = sweep_stats(v1_v)

        def body(_, carry):
            st0, st1 = carry
            g01 = gap_v[pl.ds(0, 16)]
            g01 = jnp.where(_i16() == 0, st0[2] - st0[4], g01)
            g01 = jnp.where(_i16() == 1, st1[2] - st1[4], g01)
            gap_v[pl.ds(0, 16)] = g01

            def gsweep(i, acc):
                mv, iv = acc
                c = gap_v[pl.ds(i * 16, 16)]
                idx = _i16() + i * 16
                b = c > mv
                return jnp.where(b, c, mv), jnp.where(b, idx, iv)
            mv, iv = lax.fori_loop(
                0, PCOLS // 16, gsweep,
                (jnp.full((16,), _NEG), jnp.full((16,), _BIGI)))
            gmax = jnp.max(mv)
            col = jnp.min(jnp.where(mv == gmax, iv, _BIGI))

            cbase = (col // 16) * 16
            clane = col - cbase
            mcol = _extract_i(mark_v[pl.ds(cbase, 16)], clane)
            rstat = _extract_i(row_v[pl.ds(cbase, 16)], clane)
            resel = jnp.where(mcol == 0, jnp.int32(1), jnp.int32(0))
            rother = jnp.where(mcol >= 0, resel, rstat)
            row = jnp.where(col == 0, st0[1],
                            jnp.where(col == 1, st1[1], rother))
            rbase = (row // 16) * 16
            rlane = row - rbase

            def upd(vref, st, j):
                def reset(_):
                    zero = jnp.zeros((16,), jnp.float32)

                    def zloop(i, c):
                        vref[pl.ds(i * 16, 16)] = zero
                        return c
                    lax.fori_loop(0, NQP // 16, zloop, 0)
                    ch = vref[pl.ds(rbase, 16)]
                    vref[pl.ds(rbase, 16)] = jnp.where(
                        _i16() == rlane, _MARK, ch)
                    a = jnp.where(row == 0, jnp.int32(1), jnp.int32(0))
                    z32 = jnp.float32(0.0)
                    return (z32, a, z32, a, z32)

                def knock(_):
                    ch = vref[pl.ds(rbase, 16)]
                    x = _extract_f(ch, rlane)
                    vref[pl.ds(rbase, 16)] = jnp.where(
                        _i16() == rlane, 0.0, ch)
                    m_u, a_u, m_c, a_c, s_c = st
                    need = ((row == a_u) | (row == a_c)
                            | (jnp.minimum(x, 100.0) >= s_c) | (x < 0.0))
                    return lax.cond(need, lambda __: sweep_stats(vref),
                                    lambda __: st, 0)

                return lax.cond(col == j, reset, knock, 0)

            st0 = upd(v0_v, st0, 0)
            st1 = upd(v1_v, st1, 1)

            ch = mark_v[pl.ds(cbase, 16)]
            mark_v[pl.ds(cbase, 16)] = jnp.where(
                _i16() == clane, jnp.int32(-1), ch)
            ch0 = mark_v[pl.ds(0, 16)]
            mark_v[pl.ds(0, 16)] = jnp.where(
                (_i16() <= 1) & (ch0 == row), jnp.int32(-1), ch0)
            ch = mark_v[pl.ds(cbase, 16)]
            mark_v[pl.ds(cbase, 16)] = jnp.where(_i16() == clane, row, ch)

            chg = gap_v[pl.ds(cbase, 16)]
            gap_v[pl.ds(cbase, 16)] = jnp.where(
                _i16() == clane, jnp.float32(0.0), chg)
            return st0, st1

        lax.fori_loop(0, TT, body, (st0, st1))
        pltpu.sync_copy(mark_v, mark_hbm)


@jax.jit
def kernel(pred_logits, pred_boxes, tgt_labels, tgt_boxes):
    pb = jnp.transpose(pred_boxes, (0, 2, 1))
    tb = tgt_boxes.reshape(BS, TPI, 4)
    tb = jnp.pad(tb, ((0, 0), (0, PAD - TPI), (0, 0)))
    lab = tgt_labels.reshape(BS, TPI)
    lab = jnp.pad(lab, ((0, 0), (0, PAD - TPI)))[..., None]

    gap0, rowstat, v0, v1 = pl.pallas_call(
        _phase_a,
        out_shape=[
            jax.ShapeDtypeStruct((PCOLS, 1), jnp.float32),
            jax.ShapeDtypeStruct((PCOLS, 1), jnp.int32),
            jax.ShapeDtypeStruct((1, NQ), jnp.float32),
            jax.ShapeDtypeStruct((1, NQ), jnp.float32),
        ],
        scratch_shapes=[pltpu.VMEM((PCOLS, NQ), jnp.float32)],
    )(pred_logits, pb, tb, lab)

    gap0 = gap0.reshape(PCOLS)
    rowstat = rowstat.reshape(PCOLS)
    v0 = jnp.pad(v0.reshape(NQ), (0, NQP - NQ), constant_values=_NEG)
    v1 = jnp.pad(v1.reshape(NQ), (0, NQP - NQ), constant_values=_NEG)

    mesh = plsc.VectorSubcoreMesh(core_axis_name="c", subcore_axis_name="s")
    marker = pl.kernel(
        _sc_loop,
        mesh=mesh,
        compiler_params=pltpu.CompilerParams(needs_layout_passes=False),
        out_type=jax.ShapeDtypeStruct((PCOLS,), jnp.int32),
        scratch_types=[
            pltpu.VMEM((PCOLS,), jnp.float32),
            pltpu.VMEM((PCOLS,), jnp.int32),
            pltpu.VMEM((NQP,), jnp.float32),
            pltpu.VMEM((NQP,), jnp.float32),
            pltpu.VMEM((PCOLS,), jnp.int32),
        ],
    )(gap0, rowstat, v0, v1)

    out = pl.pallas_call(
        _emit,
        out_shape=jax.ShapeDtypeStruct((NQ, PCOLS), jnp.uint8),
    )(marker.reshape(1, PCOLS))

    sel = (np.arange(TT) // TPI) * PAD + (np.arange(TT) % TPI)
    return out[:, sel].astype(jnp.bool_)


# speedup vs baseline: 1.6019x; 1.0522x over previous
import functools

import jax
import jax.numpy as jnp
import numpy as np
from jax import lax
from jax.experimental import pallas as pl
from jax.experimental.pallas import tpu as pltpu
from jax.experimental.pallas import tpu_sc as plsc

BS = 8
NQ = 900
NC = 92
TPI = 25
TT = BS * TPI
PAD = 32
PCOLS = BS * PAD
NQP = 912

_SQRT2 = np.float32(np.sqrt(np.float32(2.0)))
_DEN = np.float32(2.5) * _SQRT2
_HN = np.float32(0.3) * _SQRT2
_NEG = np.float32(-3.0e38)
_MARK = np.float32(-1e-7)
_BIGI = np.int32(2147483647)


def _phase_a(logits_ref, pb_ref, tb_ref, lab_ref,
             gap_ref, row_ref, v0_ref, v1_ref, B_ref):
    for b in range(BS):
        x = logits_ref[b]
        m = jnp.max(x, axis=1, keepdims=True)
        e = jnp.exp(x - m)
        s = jnp.sum(e, axis=1, keepdims=True)
        prob = e / s

        lab = lab_ref[b]
        onehot = (lab == jax.lax.broadcasted_iota(jnp.int32, (PAD, NC), 1)
                  ).astype(jnp.float32)
        clsprob = jax.lax.dot_general(
            onehot, prob, (((1,), (1,)), ((), ())),
            preferred_element_type=jnp.float32)

        pb = pb_ref[b]
        qcx, qcy, qw, qh = pb[0:1], pb[1:2], pb[2:3], pb[3:4]
        tb = tb_ref[b]
        tcx, tcy, tw, th = tb[:, 0:1], tb[:, 1:2], tb[:, 2:3], tb[:, 3:4]

        cbbox = (jnp.abs(qcx - tcx) + jnp.abs(qcy - tcy)
                 + jnp.abs(qw - tw) + jnp.abs(qh - th))

        qx0, qx1 = qcx - 0.5 * qw, qcx + 0.5 * qw
        qy0, qy1 = qcy - 0.5 * qh, qcy + 0.5 * qh
        tx0, tx1 = tcx - 0.5 * tw, tcx + 0.5 * tw
        ty0, ty1 = tcy - 0.5 * th, tcy + 0.5 * th

        area_q = (qx1 - qx0) * (qy1 - qy0)
        area_t = (tx1 - tx0) * (ty1 - ty0)
        iw = jnp.clip(jnp.minimum(qx1, tx1) - jnp.maximum(qx0, tx0), 0.0, None)
        ih = jnp.clip(jnp.minimum(qy1, ty1) - jnp.maximum(qy0, ty0), 0.0, None)
        inter = iw * ih
        union = area_q + area_t - inter
        iou = inter / union
        ew = jnp.clip(jnp.maximum(qx1, tx1) - jnp.minimum(qx0, tx0), 0.0, None)
        eh = jnp.clip(jnp.maximum(qy1, ty1) - jnp.minimum(qy0, ty0), 0.0, None)
        earea = ew * eh
        giou = iou - (earea - union) / earea

        C = 5.0 * cbbox + 1.0 * (-clsprob) + 2.0 * (-giou)

        z = (-C - (-5.5)) / _DEN
        p = 0.5 * (1.0 + jax.lax.erf(z))
        fxa = _HN * jax.lax.erf_inv(p)
        B_ref[PAD * b:PAD * (b + 1), :] = fxa

    col_iota = jax.lax.broadcasted_iota(jnp.int32, (PCOLS, 1), 0)
    q_iota = jax.lax.broadcasted_iota(jnp.int32, (PCOLS, NQ), 1)
    pad_col = (col_iota % PAD) >= TPI

    B = B_ref[...]
    Bc = jnp.minimum(B, 100.0)
    max1 = jnp.max(Bc, axis=1, keepdims=True)
    amax_c = jnp.min(jnp.where(Bc == max1, q_iota, NQ),
                     axis=1, keepdims=True)
    second = jnp.max(jnp.where(q_iota == amax_c, _NEG, Bc),
                     axis=1, keepdims=True)
    gap_ref[...] = jnp.where(pad_col, -1.0, max1 - second)
    maxu = jnp.max(B, axis=1, keepdims=True)
    row_ref[...] = jnp.min(jnp.where(B == maxu, q_iota, NQ),
                           axis=1, keepdims=True)
    v0_ref[...] = B_ref[0:1, :]
    v1_ref[...] = B_ref[1:2, :]


def _emit(mark_ref, out_ref):
    q_sub = jax.lax.broadcasted_iota(jnp.int32, (NQ, PCOLS), 0)
    out_ref[...] = (q_sub == mark_ref[...]).astype(jnp.uint8)


def _i16():
    return jax.lax.broadcasted_iota(jnp.int32, (16,), 0)


def _extract_f(chunk, lane):
    return jnp.max(jnp.where(_i16() == lane, chunk, _NEG))


def _extract_i(chunk, lane):
    return jnp.max(jnp.where(_i16() == lane, chunk,
                             jnp.int32(-2147483647)))


def _sc_loop(gap_hbm, row_hbm, v0_hbm, v1_hbm, mark_hbm,
             gap_v, row_v, v0_v, v1_v, mark_v):
    @pl.when((lax.axis_index("c") == 0) & (lax.axis_index("s") == 0))
    def _():
        pltpu.sync_copy(gap_hbm, gap_v)
        pltpu.sync_copy(row_hbm, row_v)
        pltpu.sync_copy(v0_hbm, v0_v)
        pltpu.sync_copy(v1_hbm, v1_v)

        neg1 = jnp.full((16,), -1, jnp.int32)

        def initm(i, carry):
            mark_v[pl.ds(i * 16, 16)] = neg1
            return carry
        lax.fori_loop(0, PCOLS // 16, initm, 0)

        def sweep_stats(vref):
            def p1(i, carry):
                mu, iu, mc, ic = carry
                c = vref[pl.ds(i * 16, 16)]
                cc = jnp.minimum(c, 100.0)
                idx = _i16() + i * 16
                bu = c > mu
                mu = jnp.where(bu, c, mu)
                iu = jnp.where(bu, idx, iu)
                bc = cc > mc
                mc = jnp.where(bc, cc, mc)
                ic = jnp.where(bc, idx, ic)
                return mu, iu, mc, ic
            init = (jnp.full((16,), _NEG), jnp.full((16,), _BIGI),
                    jnp.full((16,), _NEG), jnp.full((16,), _BIGI))
            mu, iu, mc, ic = lax.fori_loop(0, NQP // 16, p1, init)
            m_u = jnp.max(mu)
            a_u = jnp.min(jnp.where(mu == m_u, iu, _BIGI))
            m_c = jnp.max(mc)
            a_c = jnp.min(jnp.where(mc == m_c, ic, _BIGI))

            def p2(i, sv):
                c = jnp.minimum(vref[pl.ds(i * 16, 16)], 100.0)
                idx = _i16() + i * 16
                c = jnp.where(idx == a_c, _NEG, c)
                return jnp.maximum(sv, c)
            sv = lax.fori_loop(0, NQP // 16, p2, jnp.full((16,), _NEG))
            s_c = jnp.max(sv)
            return m_u, a_u, m_c, a_c, s_c

        st0 = sweep_stats(v0_v)
        st1 = sweep_stats(v1_v)

        cmax = jnp.full((16,), _NEG)
        for k in range(PCOLS // 16):
            cmax = jnp.where(_i16() == k,
                             jnp.max(gap_v[pl.ds(k * 16, 16)]), cmax)

        def body(_, carry):
            st0, st1, cmax = carry
            g01 = gap_v[pl.ds(0, 16)]
            g01 = jnp.where(_i16() == 0, st0[2] - st0[4], g01)
            g01 = jnp.where(_i16() == 1, st1[2] - st1[4], g01)
            gap_v[pl.ds(0, 16)] = g01
            cmax = jnp.where(_i16() == 0, jnp.max(g01), cmax)

            gmax = jnp.max(cmax)
            kch = jnp.min(jnp.where(cmax == gmax, _i16(), _BIGI))
            cbase = kch * 16
            gch = gap_v[pl.ds(cbase, 16)]
            clane = jnp.min(jnp.where(gch == gmax, _i16(), _BIGI))
            col = cbase + clane
            mcol = _extract_i(mark_v[pl.ds(cbase, 16)], clane)
            rstat = _extract_i(row_v[pl.ds(cbase, 16)], clane)
            resel = jnp.where(mcol == 0, jnp.int32(1), jnp.int32(0))
            rother = jnp.where(mcol >= 0, resel, rstat)
            row = jnp.where(col == 0, st0[1],
                            jnp.where(col == 1, st1[1], rother))
            rbase = (row // 16) * 16
            rlane = row - rbase

            def upd(vref, st, j):
                def reset(_):
                    zero = jnp.zeros((16,), jnp.float32)

                    def zloop(i, c):
                        vref[pl.ds(i * 16, 16)] = zero
                        return c
                    lax.fori_loop(0, NQP // 16, zloop, 0)
                    ch = vref[pl.ds(rbase, 16)]
                    vref[pl.ds(rbase, 16)] = jnp.where(
                        _i16() == rlane, _MARK, ch)
                    a = jnp.where(row == 0, jnp.int32(1), jnp.int32(0))
                    z32 = jnp.float32(0.0)
                    return (z32, a, z32, a, z32)

                def knock(_):
                    ch = vref[pl.ds(rbase, 16)]
                    x = _extract_f(ch, rlane)
                    vref[pl.ds(rbase, 16)] = jnp.where(
                        _i16() == rlane, 0.0, ch)
                    m_u, a_u, m_c, a_c, s_c = st
                    need = ((row == a_u) | (row == a_c)
                            | (jnp.minimum(x, 100.0) >= s_c) | (x < 0.0))
                    return lax.cond(need, lambda __: sweep_stats(vref),
                                    lambda __: st, 0)

                return lax.cond(col == j, reset, knock, 0)

            st0 = upd(v0_v, st0, 0)
            st1 = upd(v1_v, st1, 1)

            ch0 = mark_v[pl.ds(0, 16)]
            mark_v[pl.ds(0, 16)] = jnp.where(
                (_i16() <= 1) & (ch0 == row), jnp.int32(-1), ch0)
            ch = mark_v[pl.ds(cbase, 16)]
            mark_v[pl.ds(cbase, 16)] = jnp.where(_i16() == clane, row, ch)

            chg = gap_v[pl.ds(cbase, 16)]
            chg = jnp.where(_i16() == clane, jnp.float32(0.0), chg)
            gap_v[pl.ds(cbase, 16)] = chg
            cmax = jnp.where(_i16() == kch, jnp.max(chg), cmax)
            return st0, st1, cmax

        lax.fori_loop(0, TT, body, (st0, st1, cmax))
        pltpu.sync_copy(mark_v, mark_hbm)


@jax.jit
def kernel(pred_logits, pred_boxes, tgt_labels, tgt_boxes):
    pb = jnp.transpose(pred_boxes, (0, 2, 1))
    tb = tgt_boxes.reshape(BS, TPI, 4)
    tb = jnp.pad(tb, ((0, 0), (0, PAD - TPI), (0, 0)))
    lab = tgt_labels.reshape(BS, TPI)
    lab = jnp.pad(lab, ((0, 0), (0, PAD - TPI)))[..., None]

    gap0, rowstat, v0, v1 = pl.pallas_call(
        _phase_a,
        out_shape=[
            jax.ShapeDtypeStruct((PCOLS, 1), jnp.float32),
            jax.ShapeDtypeStruct((PCOLS, 1), jnp.int32),
            jax.ShapeDtypeStruct((1, NQ), jnp.float32),
            jax.ShapeDtypeStruct((1, NQ), jnp.float32),
        ],
        scratch_shapes=[pltpu.VMEM((PCOLS, NQ), jnp.float32)],
    )(pred_logits, pb, tb, lab)

    gap0 = gap0.reshape(PCOLS)
    rowstat = rowstat.reshape(PCOLS)
    v0 = jnp.pad(v0.reshape(NQ), (0, NQP - NQ), constant_values=_NEG)
    v1 = jnp.pad(v1.reshape(NQ), (0, NQP - NQ), constant_values=_NEG)

    mesh = plsc.VectorSubcoreMesh(core_axis_name="c", subcore_axis_name="s")
    marker = pl.kernel(
        _sc_loop,
        mesh=mesh,
        compiler_params=pltpu.CompilerParams(needs_layout_passes=False),
        out_type=jax.ShapeDtypeStruct((PCOLS,), jnp.int32),
        scratch_types=[
            pltpu.VMEM((PCOLS,), jnp.float32),
            pltpu.VMEM((PCOLS,), jnp.int32),
            pltpu.VMEM((NQP,), jnp.float32),
            pltpu.VMEM((NQP,), jnp.float32),
            pltpu.VMEM((PCOLS,), jnp.int32),
        ],
    )(gap0, rowstat, v0, v1)

    out = pl.pallas_call(
        _emit,
        out_shape=jax.ShapeDtypeStruct((NQ, PCOLS), jnp.uint8),
    )(marker.reshape(1, PCOLS))

    sel = (np.arange(TT) // TPI) * PAD + (np.arange(TT) % TPI)
    return out[:, sel].astype(jnp.bool_)


# speedup vs baseline: 1.6240x; 1.0138x over previous
import functools

import jax
import jax.numpy as jnp
import numpy as np
from jax import lax
from jax.experimental import pallas as pl
from jax.experimental.pallas import tpu as pltpu
from jax.experimental.pallas import tpu_sc as plsc

BS = 8
NQ = 900
NC = 92
TPI = 25
TT = BS * TPI
PAD = 32
PCOLS = BS * PAD
NQP = 912

_SQRT2 = np.float32(np.sqrt(np.float32(2.0)))
_DEN = np.float32(2.5) * _SQRT2
_HN = np.float32(0.3) * _SQRT2
_NEG = np.float32(-3.0e38)
_MARK = np.float32(-1e-7)
_BIGI = np.int32(2147483647)


def _phase_a(logits_ref, pb_ref, tb_ref, lab_ref,
             gap_ref, row_ref, v0_ref, v1_ref, B_ref):
    for b in range(BS):
        x = logits_ref[b]
        m = jnp.max(x, axis=1, keepdims=True)
        e = jnp.exp(x - m)
        s = jnp.sum(e, axis=1, keepdims=True)
        prob = e / s

        lab = lab_ref[b]
        onehot = (lab == jax.lax.broadcasted_iota(jnp.int32, (PAD, NC), 1)
                  ).astype(jnp.float32)
        clsprob = jax.lax.dot_general(
            onehot, prob, (((1,), (1,)), ((), ())),
            preferred_element_type=jnp.float32)

        pb = pb_ref[b]
        qcx, qcy, qw, qh = pb[0:1], pb[1:2], pb[2:3], pb[3:4]
        tb = tb_ref[b]
        tcx, tcy, tw, th = tb[:, 0:1], tb[:, 1:2], tb[:, 2:3], tb[:, 3:4]

        cbbox = (jnp.abs(qcx - tcx) + jnp.abs(qcy - tcy)
                 + jnp.abs(qw - tw) + jnp.abs(qh - th))

        qx0, qx1 = qcx - 0.5 * qw, qcx + 0.5 * qw
        qy0, qy1 = qcy - 0.5 * qh, qcy + 0.5 * qh
        tx0, tx1 = tcx - 0.5 * tw, tcx + 0.5 * tw
        ty0, ty1 = tcy - 0.5 * th, tcy + 0.5 * th

        area_q = (qx1 - qx0) * (qy1 - qy0)
        area_t = (tx1 - tx0) * (ty1 - ty0)
        iw = jnp.clip(jnp.minimum(qx1, tx1) - jnp.maximum(qx0, tx0), 0.0, None)
        ih = jnp.clip(jnp.minimum(qy1, ty1) - jnp.maximum(qy0, ty0), 0.0, None)
        inter = iw * ih
        union = area_q + area_t - inter
        iou = inter / union
        ew = jnp.clip(jnp.maximum(qx1, tx1) - jnp.minimum(qx0, tx0), 0.0, None)
        eh = jnp.clip(jnp.maximum(qy1, ty1) - jnp.minimum(qy0, ty0), 0.0, None)
        earea = ew * eh
        giou = iou - (earea - union) / earea

        C = 5.0 * cbbox + 1.0 * (-clsprob) + 2.0 * (-giou)

        z = (-C - (-5.5)) / _DEN
        p = 0.5 * (1.0 + jax.lax.erf(z))
        fxa = _HN * jax.lax.erf_inv(p)
        B_ref[PAD * b:PAD * (b + 1), :] = fxa

    col_iota = jax.lax.broadcasted_iota(jnp.int32, (PCOLS, 1), 0)
    q_iota = jax.lax.broadcasted_iota(jnp.int32, (PCOLS, NQ), 1)
    pad_col = (col_iota % PAD) >= TPI

    B = B_ref[...]
    Bc = jnp.minimum(B, 100.0)
    max1 = jnp.max(Bc, axis=1, keepdims=True)
    amax_c = jnp.min(jnp.where(Bc == max1, q_iota, NQ),
                     axis=1, keepdims=True)
    second = jnp.max(jnp.where(q_iota == amax_c, _NEG, Bc),
                     axis=1, keepdims=True)
    gap_ref[...] = jnp.where(pad_col, -1.0, max1 - second)
    maxu = jnp.max(B, axis=1, keepdims=True)
    row_ref[...] = jnp.min(jnp.where(B == maxu, q_iota, NQ),
                           axis=1, keepdims=True)
    v0_ref[0:1, 0:NQ] = B_ref[0:1, :]
    v0_ref[0:1, NQ:NQP] = jnp.full((1, NQP - NQ), _NEG)
    v1_ref[0:1, 0:NQ] = B_ref[1:2, :]
    v1_ref[0:1, NQ:NQP] = jnp.full((1, NQP - NQ), _NEG)


def _emit(mark_ref, out_ref):
    q_sub = jax.lax.broadcasted_iota(jnp.int32, (NQ, PCOLS), 0)
    out_ref[...] = (q_sub == mark_ref[...]).astype(jnp.uint8)


def _i16():
    return jax.lax.broadcasted_iota(jnp.int32, (16,), 0)


def _extract_f(chunk, lane):
    return jnp.max(jnp.where(_i16() == lane, chunk, _NEG))


def _extract_i(chunk, lane):
    return jnp.max(jnp.where(_i16() == lane, chunk,
                             jnp.int32(-2147483647)))


def _sc_loop(gap_hbm, row_hbm, v0_hbm, v1_hbm, mark_hbm,
             gap_v, row_v, v0_v, v1_v, mark_v):
    @pl.when((lax.axis_index("c") == 0) & (lax.axis_index("s") == 0))
    def _():
        pltpu.sync_copy(gap_hbm, gap_v)
        pltpu.sync_copy(row_hbm, row_v)
        pltpu.sync_copy(v0_hbm, v0_v)
        pltpu.sync_copy(v1_hbm, v1_v)

        neg1 = jnp.full((16,), -1, jnp.int32)

        def initm(i, carry):
            mark_v[pl.ds(i * 16, 16)] = neg1
            return carry
        lax.fori_loop(0, PCOLS // 16, initm, 0)

        def sweep_stats(vref):
            def p1(i, carry):
                mu, iu, mc, ic = carry
                c = vref[pl.ds(i * 16, 16)]
                cc = jnp.minimum(c, 100.0)
                idx = _i16() + i * 16
                bu = c > mu
                mu = jnp.where(bu, c, mu)
                iu = jnp.where(bu, idx, iu)
                bc = cc > mc
                mc = jnp.where(bc, cc, mc)
                ic = jnp.where(bc, idx, ic)
                return mu, iu, mc, ic
            init = (jnp.full((16,), _NEG), jnp.full((16,), _BIGI),
                    jnp.full((16,), _NEG), jnp.full((16,), _BIGI))
            mu, iu, mc, ic = lax.fori_loop(0, NQP // 16, p1, init)
            m_u = jnp.max(mu)
            a_u = jnp.min(jnp.where(mu == m_u, iu, _BIGI))
            m_c = jnp.max(mc)
            a_c = jnp.min(jnp.where(mc == m_c, ic, _BIGI))

            def p2(i, sv):
                c = jnp.minimum(vref[pl.ds(i * 16, 16)], 100.0)
                idx = _i16() + i * 16
                c = jnp.where(idx == a_c, _NEG, c)
                return jnp.maximum(sv, c)
            sv = lax.fori_loop(0, NQP // 16, p2, jnp.full((16,), _NEG))
            s_c = jnp.max(sv)
            return m_u, a_u, m_c, a_c, s_c

        st0 = sweep_stats(v0_v)
        st1 = sweep_stats(v1_v)

        cmax = jnp.full((16,), _NEG)
        for k in range(PCOLS // 16):
            cmax = jnp.where(_i16() == k,
                             jnp.max(gap_v[pl.ds(k * 16, 16)]), cmax)

        def body(_, carry):
            st0, st1, cmax = carry
            g01 = gap_v[pl.ds(0, 16)]
            g01 = jnp.where(_i16() == 0, st0[2] - st0[4], g01)
            g01 = jnp.where(_i16() == 1, st1[2] - st1[4], g01)
            gap_v[pl.ds(0, 16)] = g01
            cmax = jnp.where(_i16() == 0, jnp.max(g01), cmax)

            gmax = jnp.max(cmax)
            kch = jnp.min(jnp.where(cmax == gmax, _i16(), _BIGI))
            cbase = kch * 16
            gch = gap_v[pl.ds(cbase, 16)]
            clane = jnp.min(jnp.where(gch == gmax, _i16(), _BIGI))
            col = cbase + clane
            chr_ = row_v[pl.ds(cbase, 16)]
            rother = _extract_i(chr_, clane)
            row = jnp.where(col == 0, st0[1],
                            jnp.where(col == 1, st1[1], rother))
            row_v[pl.ds(cbase, 16)] = jnp.where(
                _i16() == clane,
                jnp.where(row == 0, jnp.int32(1), jnp.int32(0)), chr_)
            rbase = (row // 16) * 16
            rlane = row - rbase

            def upd(vref, st, j):
                def reset(_):
                    zero = jnp.zeros((16,), jnp.float32)

                    def zloop(i, c):
                        vref[pl.ds(i * 16, 16)] = zero
                        return c
                    lax.fori_loop(0, NQP // 16, zloop, 0)
                    ch = vref[pl.ds(rbase, 16)]
                    vref[pl.ds(rbase, 16)] = jnp.where(
                        _i16() == rlane, _MARK, ch)
                    a = jnp.where(row == 0, jnp.int32(1), jnp.int32(0))
                    z32 = jnp.float32(0.0)
                    return (z32, a, z32, a, z32)

                def knock(_):
                    ch = vref[pl.ds(rbase, 16)]
                    x = _extract_f(ch, rlane)
                    vref[pl.ds(rbase, 16)] = jnp.where(
                        _i16() == rlane, 0.0, ch)
                    m_u, a_u, m_c, a_c, s_c = st
                    need = ((row == a_u) | (row == a_c)
                            | (jnp.minimum(x, 100.0) >= s_c) | (x < 0.0))
                    return lax.cond(need, lambda __: sweep_stats(vref),
                                    lambda __: st, 0)

                return lax.cond(col == j, reset, knock, 0)

            st0 = upd(v0_v, st0, 0)
            st1 = upd(v1_v, st1, 1)

            ch0 = mark_v[pl.ds(0, 16)]
            mark_v[pl.ds(0, 16)] = jnp.where(
                (_i16() <= 1) & (ch0 == row), jnp.int32(-1), ch0)
            ch = mark_v[pl.ds(cbase, 16)]
            mark_v[pl.ds(cbase, 16)] = jnp.where(_i16() == clane, row, ch)

            chg = gap_v[pl.ds(cbase, 16)]
            chg = jnp.where(_i16() == clane, jnp.float32(0.0), chg)
            gap_v[pl.ds(cbase, 16)] = chg
            cmax = jnp.where(_i16() == kch, jnp.max(chg), cmax)
            return st0, st1, cmax

        lax.fori_loop(0, TT, body, (st0, st1, cmax))
        pltpu.sync_copy(mark_v, mark_hbm)


@jax.jit
def kernel(pred_logits, pred_boxes, tgt_labels, tgt_boxes):
    pb = jnp.transpose(pred_boxes, (0, 2, 1))
    tb = tgt_boxes.reshape(BS, TPI, 4)
    tb = jnp.pad(tb, ((0, 0), (0, PAD - TPI), (0, 0)))
    lab = tgt_labels.reshape(BS, TPI)
    lab = jnp.pad(lab, ((0, 0), (0, PAD - TPI)))[..., None]

    gap0, rowstat, v0, v1 = pl.pallas_call(
        _phase_a,
        out_shape=[
            jax.ShapeDtypeStruct((PCOLS, 1), jnp.float32),
            jax.ShapeDtypeStruct((PCOLS, 1), jnp.int32),
            jax.ShapeDtypeStruct((1, NQP), jnp.float32),
            jax.ShapeDtypeStruct((1, NQP), jnp.float32),
        ],
        scratch_shapes=[pltpu.VMEM((PCOLS, NQ), jnp.float32)],
    )(pred_logits, pb, tb, lab)

    gap0 = gap0.reshape(PCOLS)
    rowstat = rowstat.reshape(PCOLS)
    v0 = v0.reshape(NQP)
    v1 = v1.reshape(NQP)

    mesh = plsc.VectorSubcoreMesh(core_axis_name="c", subcore_axis_name="s")
    marker = pl.kernel(
        _sc_loop,
        mesh=mesh,
        compiler_params=pltpu.CompilerParams(needs_layout_passes=False),
        out_type=jax.ShapeDtypeStruct((PCOLS,), jnp.int32),
        scratch_types=[
            pltpu.VMEM((PCOLS,), jnp.float32),
            pltpu.VMEM((PCOLS,), jnp.int32),
            pltpu.VMEM((NQP,), jnp.float32),
            pltpu.VMEM((NQP,), jnp.float32),
            pltpu.VMEM((PCOLS,), jnp.int32),
        ],
    )(gap0, rowstat, v0, v1)

    out = pl.pallas_call(
        _emit,
        out_shape=jax.ShapeDtypeStruct((NQ, PCOLS), jnp.uint8),
    )(marker.reshape(1, PCOLS))

    sel = (np.arange(TT) // TPI) * PAD + (np.arange(TT) % TPI)
    return out[:, sel].astype(jnp.bool_)
